# baseline (speedup 1.0000x reference)
import dataclasses
import functools

import jax
import jax.numpy as jnp
from jax import lax
from jax.experimental import pallas as pl
from jax.experimental.pallas import tpu as pltpu
from jax.experimental.pallas import tpu_sc as plsc

_B, _H, _W = 2, 512, 512
_N = 32768
_C = 64
_NO8 = 8
_NC, _NS = 2, 16
_NW = _NC * _NS
_SPW = _N // _NW
_SPH = _SPW // 2
_HALO = 520
_WIN = _SPH + 2 * _HALO
_QPW = _SPW * _NO8
_QPH = _SPH * _NO8
_P2 = 1024
_NT2 = _P2 + _N + _P2
_QCH = 128
_RB = 512


def _sc_params():
    cp = pltpu.CompilerParams()
    if "needs_layout_passes" in pltpu.CompilerParams.__dataclass_fields__:
        cp = dataclasses.replace(cp, needs_layout_passes=False)
    return cp


def _vmesh():
    return plsc.VectorSubcoreMesh(
        core_axis_name="c", subcore_axis_name="s", num_cores=_NC,
        num_subcores=_NS)


def _build_nbr(idx_padded):
    @functools.partial(
        pl.kernel,
        out_type=jax.ShapeDtypeStruct((_N * _NO8,), jnp.int32),
        mesh=_vmesh(),
        scratch_types=[
            pltpu.VMEM((_WIN,), jnp.int32),
            pltpu.VMEM((_QPW,), jnp.int32),
        ],
        compiler_params=_sc_params(),
    )
    def k(idx_hbm, nbr_hbm, win_v, out_v):
        wid = lax.axis_index("s") * _NC + lax.axis_index("c")
        lane = lax.iota(jnp.int32, 16)
        for h in range(2):
            start = wid * _SPW + h * _SPH
            pltpu.sync_copy(idx_hbm.at[pl.ds(start, _WIN)], win_v)

            @pl.loop(0, _SPH, step=16)
            def _(s):
                lid = h * _SPH + s + lane
                flat = win_v[pl.ds(_HALO + s, 16)]
                cx = flat & (_W - 1)
                cy = (flat >> 9) & (_H - 1)
                m = 0
                for koff in range(9):
                    dy, dx = koff // 3 - 1, koff % 3 - 1
                    if dy == 0 and dx == 0:
                        continue
                    pos_out = lid * _NO8 + m
                    m += 1
                    t = flat + (dy * _W + dx)
                    valid = None
                    if dx == -1:
                        valid = cx > 0
                    elif dx == 1:
                        valid = cx < _W - 1
                    if dy == -1:
                        vy = cy > 0
                        valid = vy if valid is None else valid & vy
                    elif dy == 1:
                        vy = cy < _H - 1
                        valid = vy if valid is None else valid & vy
                    lo = jnp.zeros((16,), jnp.int32)
                    hi = jnp.full((16,), _WIN, jnp.int32)
                    for _i in range(11):
                        mid = (lo + hi) >> 1
                        v = plsc.load_gather(win_v, [mid])
                        cond = v < t
                        lo = jnp.where(cond, mid + 1, lo)
                        hi = jnp.where(cond, hi, mid)
                    posc = jnp.minimum(lo, _WIN - 1)
                    v = plsc.load_gather(win_v, [posc])
                    hit = valid & (v == t)
                    nbr = jnp.where(hit, lo, _WIN)
                    plsc.store_scatter(out_v, [pos_out], nbr)

        pltpu.sync_copy(out_v, nbr_hbm.at[pl.ds(wid * _QPW, _QPW)])

    return k(idx_padded)


def _gather_rows(tab, nbr):
    @functools.partial(
        pl.kernel,
        out_type=jax.ShapeDtypeStruct((_N * _NO8 * _C,), jnp.float32),
        mesh=_vmesh(),
        scratch_types=[
            pltpu.VMEM(((_WIN + 8) * _C,), jnp.float32),
            pltpu.VMEM((_QPH,), jnp.int32),
            pltpu.VMEM((_QCH * _C,), jnp.float32),
            pltpu.VMEM((_QCH * _C,), jnp.float32),
            pltpu.SemaphoreType.DMA,
            pltpu.SemaphoreType.DMA,
        ],
        compiler_params=_sc_params(),
    )
    def k(tab_hbm, nbr_hbm, g_hbm, wtab, idx_v, ob0, ob1, sem0, sem1):
        wid = lax.axis_index("s") * _NC + lax.axis_index("c")
        lane = lax.iota(jnp.int32, 16)
        zeros = jnp.zeros((16,), jnp.float32)
        for kz in range(_C // 16):
            wtab[pl.ds(_WIN * _C + kz * 16, 16)] = zeros

        def fill(q0, obuf):
            @plsc.parallel_loop(0, _QCH, step=16, unroll=8)
            def _(b):
                lv = idx_v[pl.ds(q0 + b, 16)]
                src = lv << 6
                dst = (b + lane) << 6
                for kw in range(_C):
                    v = plsc.load_gather(wtab, [src + kw])
                    plsc.store_scatter(obuf, [dst + kw], v)

        for h in range(2):
            wstart = (wid * _SPW + h * _SPH + _P2 - _HALO) * _C
            pltpu.sync_copy(tab_hbm.at[pl.ds(wstart, _WIN * _C)],
                            wtab.at[pl.ds(0, _WIN * _C)])
            qbase = wid * _QPW + h * _QPH
            pltpu.sync_copy(nbr_hbm.at[pl.ds(qbase, _QPH)], idx_v)

            @pl.loop(0, _QPH, step=2 * _QCH)
            def _(q0):
                @pl.when(q0 > 0)
                def _():
                    pltpu.make_async_copy(
                        ob0, g_hbm.at[pl.ds(0, _QCH * _C)], sem0).wait()
                    pltpu.make_async_copy(
                        ob1, g_hbm.at[pl.ds(0, _QCH * _C)], sem1).wait()

                fill(q0, ob0)
                pltpu.async_copy(
                    ob0, g_hbm.at[pl.ds((qbase + q0) * _C, _QCH * _C)], sem0)
                fill(q0 + _QCH, ob1)
                pltpu.async_copy(
                    ob1, g_hbm.at[pl.ds((qbase + q0 + _QCH) * _C, _QCH * _C)],
                    sem1)

            pltpu.make_async_copy(
                ob0, g_hbm.at[pl.ds(0, _QCH * _C)], sem0).wait()
            pltpu.make_async_copy(
                ob1, g_hbm.at[pl.ds(0, _QCH * _C)], sem1).wait()

    return k(tab, nbr)


def _conv1(g, x, w8, wc, b):
    nb = _N // _RB
    npad = _P2 // _RB

    def body(g_ref, x_ref, w8_ref, wc_ref, b_ref, o_ref):
        j = pl.program_id(0)
        a = jnp.dot(g_ref[...], w8_ref[...], preferred_element_type=jnp.float32)
        a = a + jnp.dot(x_ref[...], wc_ref[...],
                        preferred_element_type=jnp.float32)
        res = jnp.maximum(a + b_ref[...], 0.0)
        live = jnp.logical_and(j >= npad, j < nb + npad)
        o_ref[...] = jnp.where(live, res, 0.0)

    site = lambda j: (jnp.clip(j - npad, 0, nb - 1), 0)
    return pl.pallas_call(
        body,
        grid=(nb + 2 * npad,),
        in_specs=[
            pl.BlockSpec((_RB, _NO8 * _C), site),
            pl.BlockSpec((_RB, _C), site),
            pl.BlockSpec((_NO8 * _C, _C), lambda j: (0, 0)),
            pl.BlockSpec((_C, _C), lambda j: (0, 0)),
            pl.BlockSpec((1, _C), lambda j: (0, 0)),
        ],
        out_specs=pl.BlockSpec((_RB, _C), lambda j: (j, 0)),
        out_shape=jax.ShapeDtypeStruct((_NT2, _C), jnp.float32),
    )(g, x, w8, wc, b)


def _conv2(g, h_tab, x, w8, wc, b, wd):
    nb = _N // _RB
    npad = _P2 // _RB

    def body(g_ref, h_ref, x_ref, w8_ref, wc_ref, b_ref, wd_ref, o_ref):
        a = jnp.dot(g_ref[...], w8_ref[...], preferred_element_type=jnp.float32)
        a = a + jnp.dot(h_ref[...], wc_ref[...],
                        preferred_element_type=jnp.float32)
        a = jnp.maximum(a + b_ref[...], 0.0)
        r = jnp.dot(x_ref[...], wd_ref[...],
                    preferred_element_type=jnp.float32)
        o_ref[...] = jnp.maximum(a + r, 0.0)

    blk = lambda j: (j, 0)
    return pl.pallas_call(
        body,
        grid=(nb,),
        in_specs=[
            pl.BlockSpec((_RB, _NO8 * _C), blk),
            pl.BlockSpec((_RB, _C), lambda j: (j + npad, 0)),
            pl.BlockSpec((_RB, _C), blk),
            pl.BlockSpec((_NO8 * _C, _C), lambda j: (0, 0)),
            pl.BlockSpec((_C, _C), lambda j: (0, 0)),
            pl.BlockSpec((1, _C), lambda j: (0, 0)),
            pl.BlockSpec((_C, _C), lambda j: (0, 0)),
        ],
        out_specs=pl.BlockSpec((_RB, _C), blk),
        out_shape=jax.ShapeDtypeStruct((_N, _C), jnp.float32),
    )(g, h_tab, x, w8, wc, b, wd)


def _split_w(w):
    w9 = w.reshape(9, _C, _C)
    w8 = jnp.concatenate([w9[:4], w9[5:]], axis=0)
    return w8.reshape(_NO8 * _C, _C), w9[4]


def kernel(features, indices, W1, b1, W2, b2, Wd):
    idx = indices.astype(jnp.int32)
    idx_padded = jnp.concatenate([
        jnp.full((_HALO,), -1, jnp.int32),
        idx,
        jnp.full((_HALO,), jnp.int32(0x3FFFFFFF)),
    ])
    nbr = _build_nbr(idx_padded)

    x_tab = jnp.pad(features, ((_P2, _P2), (0, 0)))
    w1_8, w1_c = _split_w(W1)
    w2_8, w2_c = _split_w(W2)

    g1 = _gather_rows(x_tab.reshape(_NT2 * _C), nbr).reshape(_N, _NO8 * _C)
    out1_tab = _conv1(g1, features, w1_8, w1_c, b1.reshape(1, _C))
    g2 = _gather_rows(out1_tab.reshape(_NT2 * _C), nbr).reshape(_N, _NO8 * _C)
    return _conv2(g2, out1_tab, features, w2_8, w2_c, b2.reshape(1, _C), Wd)


# speedup vs baseline: 1.1180x; 1.1180x over previous
import dataclasses
import functools

import jax
import jax.numpy as jnp
from jax import lax
from jax.experimental import pallas as pl
from jax.experimental.pallas import tpu as pltpu
from jax.experimental.pallas import tpu_sc as plsc

_B, _H, _W = 2, 512, 512
_N = 32768
_C = 64
_NO8 = 8
_NC, _NS = 2, 16
_NW = _NC * _NS
_SPW = _N // _NW
_SPH = _SPW // 2
_HALO = 520
_WIN = _SPH + 2 * _HALO
_QPW = _SPW * _NO8
_QPH = _SPH * _NO8
_P2 = 1024
_NT2 = _P2 + _N + _P2
_QCH = 128
_RB = 512


def _sc_params():
    cp = pltpu.CompilerParams()
    if "needs_layout_passes" in pltpu.CompilerParams.__dataclass_fields__:
        cp = dataclasses.replace(cp, needs_layout_passes=False)
    return cp


def _vmesh():
    return plsc.VectorSubcoreMesh(
        core_axis_name="c", subcore_axis_name="s", num_cores=_NC,
        num_subcores=_NS)


def _build_nbr(idx_padded):
    @functools.partial(
        pl.kernel,
        out_type=jax.ShapeDtypeStruct((_N * _NO8,), jnp.int32),
        mesh=_vmesh(),
        scratch_types=[
            pltpu.VMEM((_WIN,), jnp.int32),
            pltpu.VMEM((_QPW,), jnp.int32),
        ],
        compiler_params=_sc_params(),
    )
    def k(idx_hbm, nbr_hbm, win_v, out_v):
        wid = lax.axis_index("s") * _NC + lax.axis_index("c")
        lane = lax.iota(jnp.int32, 16)
        for h in range(2):
            start = wid * _SPW + h * _SPH
            pltpu.sync_copy(idx_hbm.at[pl.ds(start, _WIN)], win_v)

            @plsc.parallel_loop(0, _SPH, step=16, unroll=2)
            def _(s):
                lid = h * _SPH + s + lane
                flat = win_v[pl.ds(_HALO + s, 16)]
                cx = flat & (_W - 1)
                cy = (flat >> 9) & (_H - 1)
                m = 0
                for koff in range(9):
                    dy, dx = koff // 3 - 1, koff % 3 - 1
                    if dy == 0 and dx == 0:
                        continue
                    pos_out = lid * _NO8 + m
                    m += 1
                    t = flat + (dy * _W + dx)
                    valid = None
                    if dx == -1:
                        valid = cx > 0
                    elif dx == 1:
                        valid = cx < _W - 1
                    if dy == -1:
                        vy = cy > 0
                        valid = vy if valid is None else valid & vy
                    elif dy == 1:
                        vy = cy < _H - 1
                        valid = vy if valid is None else valid & vy
                    lo = jnp.zeros((16,), jnp.int32)
                    hi = jnp.full((16,), _WIN, jnp.int32)
                    for _i in range(11):
                        mid = (lo + hi) >> 1
                        v = plsc.load_gather(win_v, [mid])
                        cond = v < t
                        lo = jnp.where(cond, mid + 1, lo)
                        hi = jnp.where(cond, hi, mid)
                    posc = jnp.minimum(lo, _WIN - 1)
                    v = plsc.load_gather(win_v, [posc])
                    hit = valid & (v == t)
                    nbr = jnp.where(hit, lo, _WIN)
                    plsc.store_scatter(out_v, [pos_out], nbr)

        pltpu.sync_copy(out_v, nbr_hbm.at[pl.ds(wid * _QPW, _QPW)])

    return k(idx_padded)


def _gather_rows(tab, nbr):
    @functools.partial(
        pl.kernel,
        out_type=jax.ShapeDtypeStruct((_N * _NO8 * _C,), jnp.float32),
        mesh=_vmesh(),
        scratch_types=[
            pltpu.VMEM(((_WIN + 8) * _C,), jnp.float32),
            pltpu.VMEM((_QPH,), jnp.int32),
            pltpu.VMEM((_QCH * _C,), jnp.float32),
            pltpu.VMEM((_QCH * _C,), jnp.float32),
            pltpu.SemaphoreType.DMA,
            pltpu.SemaphoreType.DMA,
        ],
        compiler_params=_sc_params(),
    )
    def k(tab_hbm, nbr_hbm, g_hbm, wtab, idx_v, ob0, ob1, sem0, sem1):
        wid = lax.axis_index("s") * _NC + lax.axis_index("c")
        lane = lax.iota(jnp.int32, 16)
        zeros = jnp.zeros((16,), jnp.float32)
        for kz in range(_C // 16):
            wtab[pl.ds(_WIN * _C + kz * 16, 16)] = zeros

        def fill(q0, obuf):
            @plsc.parallel_loop(0, _QCH, step=16, unroll=4)
            def _(b):
                lv = idx_v[pl.ds(q0 + b, 16)]
                src = lv << 6
                dst = (b + lane) << 6
                for kw in range(_C):
                    v = plsc.load_gather(wtab, [src + kw])
                    plsc.store_scatter(obuf, [dst + kw], v)

        for h in range(2):
            wstart = (wid * _SPW + h * _SPH + _P2 - _HALO) * _C
            pltpu.sync_copy(tab_hbm.at[pl.ds(wstart, _WIN * _C)],
                            wtab.at[pl.ds(0, _WIN * _C)])
            qbase = wid * _QPW + h * _QPH
            pltpu.sync_copy(nbr_hbm.at[pl.ds(qbase, _QPH)], idx_v)

            @pl.loop(0, _QPH, step=2 * _QCH)
            def _(q0):
                @pl.when(q0 > 0)
                def _():
                    pltpu.make_async_copy(
                        ob0, g_hbm.at[pl.ds(0, _QCH * _C)], sem0).wait()
                    pltpu.make_async_copy(
                        ob1, g_hbm.at[pl.ds(0, _QCH * _C)], sem1).wait()

                fill(q0, ob0)
                pltpu.async_copy(
                    ob0, g_hbm.at[pl.ds((qbase + q0) * _C, _QCH * _C)], sem0)
                fill(q0 + _QCH, ob1)
                pltpu.async_copy(
                    ob1, g_hbm.at[pl.ds((qbase + q0 + _QCH) * _C, _QCH * _C)],
                    sem1)

            pltpu.make_async_copy(
                ob0, g_hbm.at[pl.ds(0, _QCH * _C)], sem0).wait()
            pltpu.make_async_copy(
                ob1, g_hbm.at[pl.ds(0, _QCH * _C)], sem1).wait()

    return k(tab, nbr)


def _conv1(g, x, w8, wc, b):
    nb = _N // _RB
    npad = _P2 // _RB

    def body(g_ref, x_ref, w8_ref, wc_ref, b_ref, o_ref):
        j = pl.program_id(0)
        a = jnp.dot(g_ref[...], w8_ref[...], preferred_element_type=jnp.float32)
        a = a + jnp.dot(x_ref[...], wc_ref[...],
                        preferred_element_type=jnp.float32)
        res = jnp.maximum(a + b_ref[...], 0.0)
        live = jnp.logical_and(j >= npad, j < nb + npad)
        o_ref[...] = jnp.where(live, res, 0.0)

    site = lambda j: (jnp.clip(j - npad, 0, nb - 1), 0)
    return pl.pallas_call(
        body,
        grid=(nb + 2 * npad,),
        in_specs=[
            pl.BlockSpec((_RB, _NO8 * _C), site),
            pl.BlockSpec((_RB, _C), site),
            pl.BlockSpec((_NO8 * _C, _C), lambda j: (0, 0)),
            pl.BlockSpec((_C, _C), lambda j: (0, 0)),
            pl.BlockSpec((1, _C), lambda j: (0, 0)),
        ],
        out_specs=pl.BlockSpec((_RB, _C), lambda j: (j, 0)),
        out_shape=jax.ShapeDtypeStruct((_NT2, _C), jnp.float32),
    )(g, x, w8, wc, b)


def _conv2(g, h_tab, x, w8, wc, b, wd):
    nb = _N // _RB
    npad = _P2 // _RB

    def body(g_ref, h_ref, x_ref, w8_ref, wc_ref, b_ref, wd_ref, o_ref):
        a = jnp.dot(g_ref[...], w8_ref[...], preferred_element_type=jnp.float32)
        a = a + jnp.dot(h_ref[...], wc_ref[...],
                        preferred_element_type=jnp.float32)
        a = jnp.maximum(a + b_ref[...], 0.0)
        r = jnp.dot(x_ref[...], wd_ref[...],
                    preferred_element_type=jnp.float32)
        o_ref[...] = jnp.maximum(a + r, 0.0)

    blk = lambda j: (j, 0)
    return pl.pallas_call(
        body,
        grid=(nb,),
        in_specs=[
            pl.BlockSpec((_RB, _NO8 * _C), blk),
            pl.BlockSpec((_RB, _C), lambda j: (j + npad, 0)),
            pl.BlockSpec((_RB, _C), blk),
            pl.BlockSpec((_NO8 * _C, _C), lambda j: (0, 0)),
            pl.BlockSpec((_C, _C), lambda j: (0, 0)),
            pl.BlockSpec((1, _C), lambda j: (0, 0)),
            pl.BlockSpec((_C, _C), lambda j: (0, 0)),
        ],
        out_specs=pl.BlockSpec((_RB, _C), blk),
        out_shape=jax.ShapeDtypeStruct((_N, _C), jnp.float32),
    )(g, h_tab, x, w8, wc, b, wd)


def _split_w(w):
    w9 = w.reshape(9, _C, _C)
    w8 = jnp.concatenate([w9[:4], w9[5:]], axis=0)
    return w8.reshape(_NO8 * _C, _C), w9[4]


def kernel(features, indices, W1, b1, W2, b2, Wd):
    idx = indices.astype(jnp.int32)
    idx_padded = jnp.concatenate([
        jnp.full((_HALO,), -1, jnp.int32),
        idx,
        jnp.full((_HALO,), jnp.int32(0x3FFFFFFF)),
    ])
    nbr = _build_nbr(idx_padded)

    x_tab = jnp.pad(features, ((_P2, _P2), (0, 0)))
    w1_8, w1_c = _split_w(W1)
    w2_8, w2_c = _split_w(W2)

    g1 = _gather_rows(x_tab.reshape(_NT2 * _C), nbr).reshape(_N, _NO8 * _C)
    out1_tab = _conv1(g1, features, w1_8, w1_c, b1.reshape(1, _C))
    g2 = _gather_rows(out1_tab.reshape(_NT2 * _C), nbr).reshape(_N, _NO8 * _C)
    return _conv2(g2, out1_tab, features, w2_8, w2_c, b2.reshape(1, _C), Wd)
